# baseline (speedup 1.0000x reference)
import functools
import math

import jax
import jax.numpy as jnp
from jax import lax
from jax.experimental import pallas as pl
from jax.experimental.pallas import tpu as pltpu
from jax.experimental.pallas import tpu_sc as plsc

D_MODEL = 64
SCALE = math.sqrt(D_MODEL)
CHUNK = 128


@functools.partial(jax.jit, static_argnames=("n_rows",))
def _embed_lookup(ids2d, table, n_rows):
    n_chunks_total = ids2d.shape[0]
    info = plsc.get_sparse_core_info()
    nw = info.num_cores * info.num_subcores
    chunks_per_w = n_chunks_total // nw

    mesh = plsc.VectorSubcoreMesh(core_axis_name="c", subcore_axis_name="s")

    @functools.partial(
        pl.kernel,
        mesh=mesh,
        out_type=jax.ShapeDtypeStruct((n_rows, D_MODEL), jnp.float32),
        scratch_types=[
            pltpu.VMEM((chunks_per_w, CHUNK), jnp.int32),
            pltpu.VMEM((CHUNK, D_MODEL), jnp.float32),
            pltpu.SemaphoreType.DMA,
        ],
        compiler_params=pltpu.CompilerParams(use_tc_tiling_on_sc=False),
    )
    def body(table_hbm, ids_hbm, out_hbm, idx_v, rows_v, sem):
        wid = lax.axis_index("s") * info.num_cores + lax.axis_index("c")
        chunk0 = wid * chunks_per_w
        pltpu.sync_copy(ids_hbm.at[pl.ds(chunk0, chunks_per_w)], idx_v)

        def chunk_body(j, _):
            pltpu.async_copy(table_hbm.at[idx_v.at[j]], rows_v, sem).wait()

            def scale_row(r, _):
                for c in range(D_MODEL // 16):
                    sl = pl.ds(c * 16, 16)
                    rows_v[r, sl] = rows_v[r, sl] * SCALE
                return 0

            lax.fori_loop(0, CHUNK, scale_row, 0)
            pltpu.sync_copy(
                rows_v, out_hbm.at[pl.ds((chunk0 + j) * CHUNK, CHUNK)]
            )
            return 0

        lax.fori_loop(0, chunks_per_w, chunk_body, 0)

    return body(table, ids2d)


def kernel(input, table):
    b, s = input.shape
    n_rows = b * s
    ids2d = input.reshape(n_rows // CHUNK, CHUNK)
    out = _embed_lookup(ids2d, table, n_rows)
    return out.reshape(b, s, D_MODEL)


# speedup vs baseline: 1.2086x; 1.2086x over previous
import functools
import math

import jax
import jax.numpy as jnp
from jax import lax
from jax.experimental import pallas as pl
from jax.experimental.pallas import tpu as pltpu
from jax.experimental.pallas import tpu_sc as plsc

D_MODEL = 64
SCALE = math.sqrt(D_MODEL)
CHUNK = 128
NBUF = 4
ROW_UNROLL = 4


@functools.partial(jax.jit, static_argnames=("n_rows",))
def _embed_lookup(ids2d, table, n_rows):
    n_chunks_total = ids2d.shape[0]
    info = plsc.get_sparse_core_info()
    nw = info.num_cores * info.num_subcores
    cpw = n_chunks_total // nw
    n_outer = cpw // NBUF
    assert cpw % NBUF == 0 and n_outer >= 2

    mesh = plsc.VectorSubcoreMesh(core_axis_name="c", subcore_axis_name="s")

    @functools.partial(
        pl.kernel,
        mesh=mesh,
        out_type=jax.ShapeDtypeStruct((n_rows, D_MODEL), jnp.float32),
        scratch_types=[
            pltpu.VMEM((cpw, CHUNK), jnp.int32),
            pltpu.VMEM((NBUF, CHUNK, D_MODEL), jnp.float32),
            pltpu.VMEM((2, CHUNK, D_MODEL), jnp.float32),
            [pltpu.SemaphoreType.DMA] * NBUF,
            [pltpu.SemaphoreType.DMA] * 2,
        ],
        compiler_params=pltpu.CompilerParams(use_tc_tiling_on_sc=False),
    )
    def body(table_hbm, ids_hbm, out_hbm, idx_v, g_bufs, o_bufs, gsems, osems):
        wid = lax.axis_index("s") * info.num_cores + lax.axis_index("c")
        chunk0 = wid * cpw
        pltpu.sync_copy(ids_hbm.at[pl.ds(chunk0, cpw)], idx_v)

        def issue_gather(j, b):
            pltpu.async_copy(table_hbm.at[idx_v.at[j]], g_bufs.at[b], gsems[b])

        def wait_gather(j, b):
            pltpu.make_async_copy(
                table_hbm.at[idx_v.at[j]], g_bufs.at[b], gsems[b]
            ).wait()

        def out_slice(j):
            return out_hbm.at[pl.ds((chunk0 + j) * CHUNK, CHUNK)]

        def issue_out(j, ob):
            pltpu.async_copy(o_bufs.at[ob], out_slice(j), osems[ob])

        def wait_out(j, ob):
            pltpu.make_async_copy(o_bufs.at[ob], out_slice(j), osems[ob]).wait()

        def scale(b, ob):
            src = g_bufs.at[b]
            dst = o_bufs.at[ob]

            def rows(r0, _):
                for ru in range(ROW_UNROLL):
                    for c in range(D_MODEL // 16):
                        sl = pl.ds(c * 16, 16)
                        dst[r0 + ru, sl] = src[r0 + ru, sl] * SCALE
                return r0 + ROW_UNROLL

            lax.fori_loop(0, CHUNK // ROW_UNROLL, lambda i, r0: rows(r0, None),
                          0)

        for b in range(NBUF):
            issue_gather(b, b)

        for b in range(NBUF):
            wait_gather(b, b)
            if b >= 2:
                wait_out(b - 2, b % 2)
            scale(b, b % 2)
            issue_gather(NBUF + b, b)
            issue_out(b, b % 2)

        def group(g, _):
            for b in range(NBUF):
                j = g * NBUF + b
                wait_gather(j, b)
                wait_out(j - 2, b % 2)
                scale(b, b % 2)
                issue_gather(j + NBUF, b)
                issue_out(j, b % 2)
            return 0

        lax.fori_loop(1, n_outer - 1, group, 0)

        for b in range(NBUF):
            j = (n_outer - 1) * NBUF + b
            wait_gather(j, b)
            wait_out(j - 2, b % 2)
            scale(b, b % 2)
            issue_out(j, b % 2)

        wait_out(cpw - 2, 0)
        wait_out(cpw - 1, 1)

    return body(table, ids2d)


def kernel(input, table):
    b, s = input.shape
    n_rows = b * s
    ids2d = input.reshape(n_rows // CHUNK, CHUNK)
    out = _embed_lookup(ids2d, table, n_rows)
    return out.reshape(b, s, D_MODEL)


# speedup vs baseline: 1.4707x; 1.2169x over previous
import functools
import math

import jax
import jax.numpy as jnp
from jax import lax
from jax.experimental import pallas as pl
from jax.experimental.pallas import tpu as pltpu
from jax.experimental.pallas import tpu_sc as plsc

D_MODEL = 64
SCALE = math.sqrt(D_MODEL)
RAW = 128
CHUNK = 128
NBUF = 4
ROW_UNROLL = 4


def _embed_lookup(ids1d, tpad):
    n_rows = ids1d.shape[0]
    info = plsc.get_sparse_core_info()
    nw = info.num_cores * info.num_subcores
    ipw = n_rows // nw
    n_chunks = ipw // CHUNK
    n_outer = n_chunks // NBUF
    assert n_chunks % NBUF == 0 and n_outer >= 2

    mesh = plsc.VectorSubcoreMesh(core_axis_name="c", subcore_axis_name="s")

    @functools.partial(
        pl.kernel,
        mesh=mesh,
        out_type=jax.ShapeDtypeStruct((n_rows, D_MODEL), jnp.float32),
        scratch_types=[
            pltpu.VMEM((ipw,), jnp.int32),
            pltpu.VMEM((NBUF, CHUNK, RAW), jnp.float32),
            pltpu.VMEM((2, CHUNK, D_MODEL), jnp.float32),
            [pltpu.SemaphoreType.DMA] * NBUF,
            [pltpu.SemaphoreType.DMA] * 2,
        ],
    )
    def body(tpad_hbm, ids_hbm, out_hbm, idx_v, g_bufs, o_bufs, gsems, osems):
        wid = lax.axis_index("s") * info.num_cores + lax.axis_index("c")
        idx0 = wid * ipw
        pltpu.sync_copy(ids_hbm.at[pl.ds(idx0, ipw)], idx_v)

        def gather_args(j, b):
            idx = idx_v.at[pl.ds(j * CHUNK, CHUNK)]
            return tpad_hbm.at[idx], g_bufs.at[b]

        def issue_gather(j, b):
            src, dst = gather_args(j, b)
            pltpu.async_copy(src, dst, gsems[b])

        def wait_gather(j, b):
            src, dst = gather_args(j, b)
            pltpu.make_async_copy(src, dst, gsems[b]).wait()

        def out_args(j, b):
            src = o_bufs.at[b % 2]
            return src, out_hbm.at[pl.ds((idx0 + j * CHUNK), CHUNK)]

        def issue_out(j, b):
            src, dst = out_args(j, b)
            pltpu.async_copy(src, dst, osems[b % 2])

        def wait_out(j, b):
            src, dst = out_args(j, b)
            pltpu.make_async_copy(src, dst, osems[b % 2]).wait()

        def scale(b):
            src = g_bufs.at[b]
            dst = o_bufs.at[b % 2]

            def rows(i, r0):
                for ru in range(ROW_UNROLL):
                    for c in range(D_MODEL // 16):
                        sl = pl.ds(c * 16, 16)
                        dst[r0 + ru, sl] = src[r0 + ru, sl] * SCALE
                return r0 + ROW_UNROLL

            lax.fori_loop(0, CHUNK // ROW_UNROLL, rows, 0)

        for b in range(NBUF):
            issue_gather(b, b)

        for b in range(NBUF):
            wait_gather(b, b)
            if b >= 2:
                wait_out(b - 2, b - 2)
            scale(b)
            issue_gather(NBUF + b, b)
            issue_out(b, b)

        def group(g, _):
            for b in range(NBUF):
                j = g * NBUF + b
                wait_gather(j, b)
                wait_out(j - 2, b - 2 if b >= 2 else b + 2)
                scale(b)
                issue_gather(j + NBUF, b)
                issue_out(j, b)
            return 0

        lax.fori_loop(1, n_outer - 1, group, 0)

        for b in range(NBUF):
            j = (n_outer - 1) * NBUF + b
            wait_gather(j, b)
            wait_out(j - 2, b - 2 if b >= 2 else b + 2)
            scale(b)
            issue_out(j, b)

        wait_out(n_chunks - 2, 2)
        wait_out(n_chunks - 1, 3)

    return body(tpad, ids1d)


def kernel(input, table):
    b, s = input.shape
    tpad = jnp.pad(table, ((0, 0), (0, RAW - D_MODEL)))
    out = _embed_lookup(input.reshape(-1), tpad)
    return out.reshape(b, s, D_MODEL)
